# Initial kernel scaffold; baseline (speedup 1.0000x reference)
import jax
import jax.numpy as jnp
from jax.experimental import pallas as pl


def kernel(rij_unit, radial_ij, first_atom_idx, lambda_weights, lxlylz, lxlylz_sum, fact_norm, z, r_idx, nat):
    raise NotImplementedError("write your pallas kernel here")


# speedup vs baseline: 12.9156x; 12.9156x over previous
import jax
import jax.numpy as jnp
from jax.experimental import pallas as pl
from jax.experimental.pallas import tpu as pltpu

N_ATOMS = 10000
N_EDGES = 160000
NRAD = 16
L = 6

_INTERPRET = False

EDGE_BLOCK = 1280
NUM_BLOCKS = N_EDGES // EDGE_BLOCK


def _accum_body(rij_ref, rad_ref, ids_ref, fact_ref, s_ref, f_ref):
    blk = pl.program_id(0)

    @pl.when(blk == 0)
    def _zero():
        s_ref[...] = jnp.zeros_like(s_ref)

    x = rij_ref[:, 0:1]
    y = rij_ref[:, 1:2]
    z = rij_ref[:, 2:3]
    g = [x * x, y * y, z * z, x * y, x * z, y * z]
    rad = rad_ref[...]
    f_ref[...] = jnp.concatenate(
        [rad * (g[l] * fact_ref[0, l]) for l in range(L)], axis=1)

    def body(i, _):
        a = ids_ref[0, 0, i]
        s_ref[pl.ds(a, 1), :] += f_ref[pl.ds(i, 1), :]
        return 0

    jax.lax.fori_loop(0, EDGE_BLOCK, body, 0)


def _finish_body(s_ref, w_ref, out_ref):
    s = s_ref[...]
    s2 = s * s
    o0 = jnp.zeros((s.shape[0], NRAD), jnp.float32)
    o1 = jnp.zeros((s.shape[0], NRAD), jnp.float32)
    for l in range(L):
        blk = s2[:, l * NRAD:(l + 1) * NRAD]
        o0 = o0 + w_ref[0, l] * blk
        o1 = o1 + w_ref[1, l] * blk
    out_ref[...] = jnp.stack([o0, o1], axis=-1)


def kernel(rij_unit, radial_ij, first_atom_idx, lambda_weights, lxlylz, lxlylz_sum, fact_norm, z, r_idx, nat):
    rad_r = jnp.take(radial_ij, r_idx, axis=2)
    ids3 = first_atom_idx.reshape(NUM_BLOCKS, 1, EDGE_BLOCK)
    fact2 = fact_norm.reshape(1, L)

    s = pl.pallas_call(
        _accum_body,
        grid=(NUM_BLOCKS,),
        in_specs=[
            pl.BlockSpec((EDGE_BLOCK, 3), lambda i: (i, 0)),
            pl.BlockSpec((EDGE_BLOCK, NRAD), lambda i: (i, 0)),
            pl.BlockSpec((1, 1, EDGE_BLOCK), lambda i: (i, 0, 0), memory_space=pltpu.SMEM),
            pl.BlockSpec((1, L), lambda i: (0, 0), memory_space=pltpu.SMEM),
        ],
        out_specs=pl.BlockSpec((N_ATOMS, NRAD * L), lambda i: (0, 0)),
        out_shape=jax.ShapeDtypeStruct((N_ATOMS, NRAD * L), jnp.float32),
        scratch_shapes=[pltpu.VMEM((EDGE_BLOCK, NRAD * L), jnp.float32)],
        interpret=_INTERPRET,
    )(rij_unit, rad_r, ids3, fact2)

    norm = jnp.power(2.0, 1.0 - jnp.float32(z))
    w = norm * lambda_weights[:, None] ** lxlylz_sum[None, :].astype(jnp.float32)

    A = 400
    out = pl.pallas_call(
        _finish_body,
        grid=(N_ATOMS // A,),
        in_specs=[
            pl.BlockSpec((A, NRAD * L), lambda i: (i, 0)),
            pl.BlockSpec((2, L), lambda i: (0, 0), memory_space=pltpu.SMEM),
        ],
        out_specs=pl.BlockSpec((A, NRAD, 2), lambda i: (i, 0, 0)),
        out_shape=jax.ShapeDtypeStruct((N_ATOMS, NRAD, 2), jnp.float32),
        interpret=_INTERPRET,
    )(s, w)
    return out


# speedup vs baseline: 51.4577x; 3.9842x over previous
import jax
import jax.numpy as jnp
from jax import lax
from jax.experimental import pallas as pl
from jax.experimental.pallas import tpu as pltpu
from jax.experimental.pallas import tpu_sc as plsc

N_ATOMS = 10000
N_EDGES = 160000
NRAD = 16
L = 6

_INTERPRET = False

NC, NS = 2, 16
NW = NC * NS
G = 16
EPAD = 163840
CHUNK = EPAD // NW
BG = 64
EB = BG * G
NBLK = CHUNK // EB
NPAD = 10240
DUMMY = N_ATOMS
KA = 16
ZB = 128
TROWS = NPAD // NS


def _sc_body(rij_ref, rad_ref, ids_ref, out_ref,
             rijv, radv, idsv, stage, stageidx, bounce, sshared):
    c = lax.axis_index("c")
    s = lax.axis_index("s")
    wid = s * NC + c
    iota16 = lax.iota(jnp.int32, 16)
    zvec = jnp.zeros((16,), jnp.float32)
    dummyv = jnp.full((16,), DUMMY, jnp.int32)

    def _zb(j, _):
        row = bounce.at[j]
        for t in range(8):
            row[pl.ds(t * 16, 16)] = zvec
        return 0
    lax.fori_loop(0, ZB, _zb, 0)

    for zb in range(TROWS // ZB):
        pltpu.sync_copy(bounce, sshared.at[pl.ds(s * TROWS + zb * ZB, ZB)])
    plsc.subcore_barrier()

    def stage_row(cc, accs):
        row = stage.at[cc]
        for l in range(L):
            row[pl.ds(l * G, G)] = accs[l]

    def group_body(i0, carry):
        ids16 = idsv[pl.ds(i0, G)]
        x16 = rijv[pl.ds(i0, G)]
        y16 = rijv[pl.ds(EB + i0, G)]
        z16 = rijv[pl.ds(2 * EB + i0, G)]
        for j in range(G):
            cur, cc, idxv, a0, a1, a2, a3, a4, a5 = carry
            aid = ids16[j]
            changed = aid != cur

            @pl.when(changed)
            def _fl():
                stage_row(cc, (a0, a1, a2, a3, a4, a5))

            chg = jnp.where(changed, jnp.int32(1), jnp.int32(0))
            m = (1 - jnp.minimum(jnp.abs(iota16 - cc), 1)) * chg
            idxv = idxv * (1 - m) + cur * m
            cc2 = cc + chg
            full = cc2 == KA

            @pl.when(full)
            def _dma():
                stageidx[pl.ds(0, G)] = idxv
                pltpu.sync_copy(stage, sshared.at[stageidx], add=True)

            fullw = jnp.where(full, jnp.int32(1), jnp.int32(0))
            idxv = idxv * (1 - fullw) + DUMMY * fullw
            cc3 = cc2 * (1 - fullw)
            keep = jnp.where(changed, jnp.float32(0), jnp.float32(1))
            x = x16[j]
            y = y16[j]
            z = z16[j]
            rvec = radv[pl.ds((i0 + j) * NRAD, NRAD)]
            a0 = a0 * keep + (x * x) * rvec
            a1 = a1 * keep + (y * y) * rvec
            a2 = a2 * keep + (z * z) * rvec
            a3 = a3 * keep + (x * y) * rvec
            a4 = a4 * keep + (x * z) * rvec
            a5 = a5 * keep + (y * z) * rvec
            carry = (aid, cc3, idxv, a0, a1, a2, a3, a4, a5)
        return carry

    base_e = wid * CHUNK

    carry = (jnp.int32(DUMMY), jnp.int32(0), dummyv) + (zvec,) * L
    for b in range(NBLK):
        e0 = base_e + b * EB
        pltpu.sync_copy(rij_ref.at[pl.ds(e0, EB)], rijv.at[pl.ds(0, EB)])
        pltpu.sync_copy(rij_ref.at[pl.ds(EPAD + e0, EB)], rijv.at[pl.ds(EB, EB)])
        pltpu.sync_copy(rij_ref.at[pl.ds(2 * EPAD + e0, EB)], rijv.at[pl.ds(2 * EB, EB)])
        pltpu.sync_copy(rad_ref.at[pl.ds(e0 * NRAD, EB * NRAD)], radv)
        pltpu.sync_copy(ids_ref.at[pl.ds(e0, EB)], idsv)
        carry = lax.fori_loop(0, BG, lambda g, cr: group_body(g * G, cr), carry)

    cur, cc, idxv = carry[0], carry[1], carry[2]
    stage_row(cc, carry[3:])
    m = 1 - jnp.minimum(jnp.abs(iota16 - cc), 1)
    idxv = idxv * (1 - m) + cur * m
    stageidx[pl.ds(0, G)] = idxv
    pltpu.sync_copy(stage, sshared.at[stageidx], add=True)
    plsc.subcore_barrier()

    for zb in range(TROWS // ZB):
        row0 = s * TROWS + zb * ZB
        pltpu.sync_copy(sshared.at[pl.ds(row0, ZB)], bounce)
        pltpu.sync_copy(bounce, out_ref.at[c, pl.ds(row0, ZB)])


def _finish_body(s_ref, w_ref, out_ref):
    s = s_ref[...]
    o0 = jnp.zeros((s.shape[1], NRAD), jnp.float32)
    o1 = jnp.zeros((s.shape[1], NRAD), jnp.float32)
    for l in range(L):
        t = s[0, :, l * NRAD:(l + 1) * NRAD] + s[1, :, l * NRAD:(l + 1) * NRAD]
        t2 = t * t
        o0 = o0 + w_ref[0, l] * t2
        o1 = o1 + w_ref[1, l] * t2
    out_ref[...] = jnp.stack([o0, o1], axis=-1)


def kernel(rij_unit, radial_ij, first_atom_idx, lambda_weights, lxlylz, lxlylz_sum, fact_norm, z, r_idx, nat):
    npd = EPAD - N_EDGES
    rij_flat = jnp.concatenate(
        [rij_unit.T, jnp.zeros((3, npd), jnp.float32)], axis=1).reshape(-1)
    rad_flat = jnp.concatenate(
        [jnp.take(radial_ij, r_idx, axis=2),
         jnp.zeros((npd, NRAD), jnp.float32)], axis=0).reshape(-1)
    ids_pad = jnp.concatenate(
        [first_atom_idx, jnp.full((npd,), DUMMY, jnp.int32)])

    mesh = plsc.VectorSubcoreMesh(core_axis_name="c", subcore_axis_name="s",
                                  num_cores=NC, num_subcores=NS)
    s_part = pl.kernel(
        _sc_body,
        out_type=jax.ShapeDtypeStruct((NC, NPAD, 128), jnp.float32),
        mesh=mesh,
        scratch_types=[
            pltpu.VMEM((3 * EB,), jnp.float32),
            pltpu.VMEM((EB * NRAD,), jnp.float32),
            pltpu.VMEM((EB,), jnp.int32),
            pltpu.VMEM((KA, 128), jnp.float32),
            pltpu.VMEM((KA,), jnp.int32),
            pltpu.VMEM((ZB, 128), jnp.float32),
            pltpu.VMEM_SHARED((NPAD, 128), jnp.float32),
        ],
        interpret=_INTERPRET,
    )(rij_flat, rad_flat, ids_pad)

    norm = jnp.power(2.0, 1.0 - jnp.float32(z))
    w = (norm * lambda_weights[:, None] ** lxlylz_sum[None, :].astype(jnp.float32)
         * (fact_norm * fact_norm)[None, :])

    A = 400
    out = pl.pallas_call(
        _finish_body,
        grid=(N_ATOMS // A,),
        in_specs=[
            pl.BlockSpec((NC, A, 128), lambda i: (0, i, 0)),
            pl.BlockSpec((2, L), lambda i: (0, 0), memory_space=pltpu.SMEM),
        ],
        out_specs=pl.BlockSpec((A, NRAD, 2), lambda i: (i, 0, 0)),
        out_shape=jax.ShapeDtypeStruct((N_ATOMS, NRAD, 2), jnp.float32),
        interpret=_INTERPRET,
    )(s_part, w)
    return out
